# baseline (speedup 1.0000x reference)
import dataclasses
import functools

import jax
import jax.numpy as jnp
from jax.experimental import pallas as pl
from jax.experimental.pallas import tpu as pltpu
from jax.experimental.pallas import tpu_sc as plsc

NUM_ACT = 96
NUM_COEFFS = 256
X_MIN = -2.0
X_MAX = 2.0
GRID = (X_MAX - X_MIN) / (NUM_COEFFS - 1)
INV_GRID = 1.0 / GRID

LANES = 16
H = 224
W = 224
NROWS = 8 * NUM_ACT
BR = 56
BLOCKS_PER_IMG = H // BR


def _sc_spline(x3, coeff_flat, bases3):
    mesh = plsc.VectorSubcoreMesh(core_axis_name="c", subcore_axis_name="s")
    cp = pltpu.CompilerParams()
    if "needs_layout_passes" in pltpu.CompilerParams.__dataclass_fields__:
        cp = dataclasses.replace(cp, needs_layout_passes=False)
    cp = dataclasses.replace(cp, use_tc_tiling_on_sc=True)

    @functools.partial(
        pl.kernel,
        mesh=mesh,
        out_type=jax.ShapeDtypeStruct((NROWS, H, W), jnp.float32),
        scratch_types=[
            pltpu.VMEM((NUM_ACT * NUM_COEFFS,), jnp.float32),
            pltpu.VMEM((NUM_ACT * NUM_COEFFS,), jnp.float32),
        ],
        compiler_params=cp,
    )
    def run(x_hbm, coeff_hbm, bases_hbm, out_hbm, table_vmem, slope_vmem):
        pltpu.sync_copy(coeff_hbm, table_vmem)
        iota = jax.lax.iota(jnp.int32, LANES)
        nt = NUM_ACT * NUM_COEFFS

        @plsc.parallel_loop(0, nt, step=LANES, unroll=4)
        def _(i):
            nxt = plsc.load_gather(table_vmem, [jnp.minimum(iota + (i + 1), nt - 1)])
            slope_vmem[pl.ds(i, LANES)] = nxt - table_vmem[pl.ds(i, LANES)]

        def body(x_vmem, b_vmem, o_vmem):
            base = b_vmem[0, 0, pl.ds(0, LANES)]

            @plsc.parallel_loop(0, BR * W, step=LANES, unroll=8)
            def _(i):
                r = i // W
                c = i % W
                xv = x_vmem[0, r, pl.ds(c, LANES)]
                u = xv * INV_GRID + (-X_MIN * INV_GRID)
                idx = jnp.clip(u.astype(jnp.int32), 0, NUM_COEFFS - 2)
                frac = u - idx.astype(jnp.float32)
                g = base + idx
                c0 = plsc.load_gather(table_vmem, [g])
                s = plsc.load_gather(slope_vmem, [g])
                o_vmem[0, r, pl.ds(c, LANES)] = c0 + frac * s

        pltpu.emit_pipeline(
            body,
            grid=(NROWS, BLOCKS_PER_IMG),
            in_specs=[
                pl.BlockSpec((1, BR, W), lambda i, j: (i, j, 0)),
                pl.BlockSpec((1, 8, 128), lambda i, j: (i, 0, 0)),
            ],
            out_specs=[pl.BlockSpec((1, BR, W), lambda i, j: (i, j, 0))],
            core_axis_name=("c", "s"),
            dimension_semantics=(pltpu.PARALLEL, pltpu.PARALLEL),
        )(x_hbm, bases_hbm, out_hbm)

    return run(x3, coeff_flat, bases3)


def kernel(x, coefficients):
    x3 = x.reshape(NROWS, H, W)
    coeff_flat = coefficients.reshape(-1)
    bases = (jnp.arange(NROWS, dtype=jnp.int32) % NUM_ACT) * NUM_COEFFS
    bases3 = jnp.broadcast_to(bases[:, None, None], (NROWS, 8, 128))
    out = _sc_spline(x3, coeff_flat, bases3)
    return out.reshape(x.shape)


# speedup vs baseline: 1.0808x; 1.0808x over previous
import dataclasses
import functools

import jax
import jax.numpy as jnp
from jax.experimental import pallas as pl
from jax.experimental.pallas import tpu as pltpu
from jax.experimental.pallas import tpu_sc as plsc

NUM_ACT = 96
NUM_COEFFS = 256
X_MIN = -2.0
X_MAX = 2.0
GRID = (X_MAX - X_MIN) / (NUM_COEFFS - 1)
INV_GRID = 1.0 / GRID

LANES = 16
H = 224
W = 224
NROWS = 8 * NUM_ACT
BR = 56
BLOCKS_PER_IMG = H // BR


def _sc_spline(x3, coeff_flat, bases3):
    mesh = plsc.VectorSubcoreMesh(core_axis_name="c", subcore_axis_name="s")
    cp = pltpu.CompilerParams()
    if "needs_layout_passes" in pltpu.CompilerParams.__dataclass_fields__:
        cp = dataclasses.replace(cp, needs_layout_passes=False)
    cp = dataclasses.replace(cp, use_tc_tiling_on_sc=True)

    @functools.partial(
        pl.kernel,
        mesh=mesh,
        out_type=jax.ShapeDtypeStruct((NROWS, H, W), jnp.float32),
        scratch_types=[
            pltpu.VMEM((NUM_ACT * NUM_COEFFS,), jnp.float32),
            pltpu.VMEM((NUM_ACT * NUM_COEFFS,), jnp.float32),
        ],
        compiler_params=cp,
    )
    def run(x_hbm, coeff_hbm, bases_hbm, out_hbm, table_vmem, slope_vmem):
        pltpu.sync_copy(coeff_hbm, table_vmem)
        del slope_vmem

        def body(x_vmem, b_vmem, o_vmem):
            base = b_vmem[0, 0, pl.ds(0, LANES)]

            @plsc.parallel_loop(0, BR * W, step=LANES, unroll=8)
            def _(i):
                r = i // W
                c = i % W
                xv = x_vmem[0, r, pl.ds(c, LANES)]
                u = xv * INV_GRID + (-X_MIN * INV_GRID)
                idx = jnp.clip(u.astype(jnp.int32), 0, NUM_COEFFS - 2)
                frac = u - idx.astype(jnp.float32)
                g = base + idx
                c0 = plsc.load_gather(table_vmem, [g])
                c1 = plsc.load_gather(table_vmem, [g + 1])
                o_vmem[0, r, pl.ds(c, LANES)] = c0 + frac * (c1 - c0)

        pltpu.emit_pipeline(
            body,
            grid=(NROWS, BLOCKS_PER_IMG),
            in_specs=[
                pl.BlockSpec((1, BR, W), lambda i, j: (i, j, 0)),
                pl.BlockSpec((1, 8, 128), lambda i, j: (i, 0, 0)),
            ],
            out_specs=[pl.BlockSpec((1, BR, W), lambda i, j: (i, j, 0))],
            core_axis_name=("c", "s"),
            dimension_semantics=(pltpu.PARALLEL, pltpu.PARALLEL),
        )(x_hbm, bases_hbm, out_hbm)

    return run(x3, coeff_flat, bases3)


def kernel(x, coefficients):
    x3 = x.reshape(NROWS, H, W)
    coeff_flat = coefficients.reshape(-1)
    bases = (jnp.arange(NROWS, dtype=jnp.int32) % NUM_ACT) * NUM_COEFFS
    bases3 = jnp.broadcast_to(bases[:, None, None], (NROWS, 8, 128))
    out = _sc_spline(x3, coeff_flat, bases3)
    return out.reshape(x.shape)


# speedup vs baseline: 1.2251x; 1.1335x over previous
import dataclasses
import functools

import jax
import jax.numpy as jnp
from jax.experimental import pallas as pl
from jax.experimental.pallas import tpu as pltpu
from jax.experimental.pallas import tpu_sc as plsc

NUM_ACT = 96
NUM_COEFFS = 256
X_MIN = -2.0
X_MAX = 2.0
GRID = (X_MAX - X_MIN) / (NUM_COEFFS - 1)
INV_GRID = 1.0 / GRID

LANES = 16
H = 224
W = 224
NROWS = 8 * NUM_ACT
BR = 56
BLOCKS_PER_IMG = H // BR


def _sc_spline(x3, coeff_flat, bases3):
    mesh = plsc.VectorSubcoreMesh(core_axis_name="c", subcore_axis_name="s")
    cp = pltpu.CompilerParams()
    if "needs_layout_passes" in pltpu.CompilerParams.__dataclass_fields__:
        cp = dataclasses.replace(cp, needs_layout_passes=False)
    cp = dataclasses.replace(cp, use_tc_tiling_on_sc=True)

    @functools.partial(
        pl.kernel,
        mesh=mesh,
        out_type=jax.ShapeDtypeStruct((NROWS, H, W), jnp.float32),
        scratch_types=[pltpu.VMEM((NUM_ACT * NUM_COEFFS,), jnp.float32)],
        compiler_params=cp,
    )
    def run(x_hbm, coeff_hbm, bases_hbm, out_hbm, table_vmem):
        pltpu.sync_copy(coeff_hbm, table_vmem)

        def body(x_vmem, b_vmem, o_vmem):
            base = b_vmem[0, 0, pl.ds(0, LANES)]

            @plsc.parallel_loop(0, BR, step=1, unroll=2)
            def _(r):
                for c in range(0, W, LANES):
                    xv = x_vmem[0, r, pl.ds(c, LANES)]
                    u = (xv - X_MIN) * INV_GRID
                    t = jnp.clip(u, 0.0, float(NUM_COEFFS - 2))
                    idx = t.astype(jnp.int32)
                    frac = u - idx.astype(jnp.float32)
                    g = base + idx
                    c0 = plsc.load_gather(table_vmem, [g])
                    c1 = plsc.load_gather(table_vmem, [g + 1])
                    o_vmem[0, r, pl.ds(c, LANES)] = c0 + frac * (c1 - c0)

        pltpu.emit_pipeline(
            body,
            grid=(NROWS, BLOCKS_PER_IMG),
            in_specs=[
                pl.BlockSpec((1, BR, W), lambda i, j: (i, j, 0)),
                pl.BlockSpec((1, 8, 128), lambda i, j: (i, 0, 0)),
            ],
            out_specs=[pl.BlockSpec((1, BR, W), lambda i, j: (i, j, 0))],
            core_axis_name=("c", "s"),
            dimension_semantics=(pltpu.PARALLEL, pltpu.PARALLEL),
        )(x_hbm, bases_hbm, out_hbm)

    return run(x3, coeff_flat, bases3)


def kernel(x, coefficients):
    x3 = x.reshape(NROWS, H, W)
    coeff_flat = coefficients.reshape(-1)
    bases = (jnp.arange(NROWS, dtype=jnp.int32) % NUM_ACT) * NUM_COEFFS
    bases3 = jnp.broadcast_to(bases[:, None, None], (NROWS, 8, 128))
    out = _sc_spline(x3, coeff_flat, bases3)
    return out.reshape(x.shape)


# speedup vs baseline: 1.3096x; 1.0690x over previous
import dataclasses
import functools

import jax
import jax.numpy as jnp
from jax.experimental import pallas as pl
from jax.experimental.pallas import tpu as pltpu
from jax.experimental.pallas import tpu_sc as plsc

NUM_ACT = 96
NUM_COEFFS = 256
X_MIN = -2.0
X_MAX = 2.0
GRID = (X_MAX - X_MIN) / (NUM_COEFFS - 1)
INV_GRID = 1.0 / GRID

LANES = 16
H = 224
W = 224
NROWS = 8 * NUM_ACT
BR = 56
BLOCKS_PER_IMG = H // BR


def _sc_spline(x3, coeff_flat, bases3):
    mesh = plsc.VectorSubcoreMesh(core_axis_name="c", subcore_axis_name="s")
    cp = pltpu.CompilerParams()
    if "needs_layout_passes" in pltpu.CompilerParams.__dataclass_fields__:
        cp = dataclasses.replace(cp, needs_layout_passes=False)
    cp = dataclasses.replace(cp, use_tc_tiling_on_sc=True)

    @functools.partial(
        pl.kernel,
        mesh=mesh,
        out_type=jax.ShapeDtypeStruct((NROWS, H, W), jnp.float32),
        scratch_types=[
            pltpu.VMEM((NUM_ACT * NUM_COEFFS,), jnp.float32),
            pltpu.VMEM((NUM_ACT * NUM_COEFFS,), jnp.float32),
        ],
        compiler_params=cp,
    )
    def run(x_hbm, coeff_hbm, bases_hbm, out_hbm, table_vmem, slope_vmem):
        pltpu.sync_copy(coeff_hbm, table_vmem)
        iota = jax.lax.iota(jnp.int32, LANES)
        nt = NUM_ACT * NUM_COEFFS

        @plsc.parallel_loop(0, nt, step=LANES, unroll=4)
        def _(i):
            nxt = plsc.load_gather(table_vmem, [jnp.minimum(iota + (i + 1), nt - 1)])
            slope_vmem[pl.ds(i, LANES)] = nxt - table_vmem[pl.ds(i, LANES)]

        @plsc.parallel_loop(0, nt, step=LANES, unroll=4)
        def _(i):
            knot = ((iota + i) & (NUM_COEFFS - 1)).astype(jnp.float32)
            table_vmem[pl.ds(i, LANES)] = (
                table_vmem[pl.ds(i, LANES)] - knot * slope_vmem[pl.ds(i, LANES)]
            )

        def body(x_vmem, b_vmem, o_vmem):
            base = b_vmem[0, 0, pl.ds(0, LANES)]

            @plsc.parallel_loop(0, BR, step=1, unroll=2)
            def _(r):
                for c in range(0, W, LANES):
                    xv = x_vmem[0, r, pl.ds(c, LANES)]
                    u = (xv - X_MIN) * INV_GRID
                    t = jnp.clip(u, 0.0, float(NUM_COEFFS - 2))
                    idx = t.astype(jnp.int32)
                    g = base + idx
                    a = plsc.load_gather(table_vmem, [g])
                    b = plsc.load_gather(slope_vmem, [g])
                    o_vmem[0, r, pl.ds(c, LANES)] = a + u * b

        pltpu.emit_pipeline(
            body,
            grid=(NROWS, BLOCKS_PER_IMG),
            in_specs=[
                pl.BlockSpec((1, BR, W), lambda i, j: (i, j, 0)),
                pl.BlockSpec((1, 8, 128), lambda i, j: (i, 0, 0)),
            ],
            out_specs=[pl.BlockSpec((1, BR, W), lambda i, j: (i, j, 0))],
            core_axis_name=("c", "s"),
            dimension_semantics=(pltpu.PARALLEL, pltpu.PARALLEL),
        )(x_hbm, bases_hbm, out_hbm)

    return run(x3, coeff_flat, bases3)


def kernel(x, coefficients):
    x3 = x.reshape(NROWS, H, W)
    coeff_flat = coefficients.reshape(-1)
    bases = (jnp.arange(NROWS, dtype=jnp.int32) % NUM_ACT) * NUM_COEFFS
    bases3 = jnp.broadcast_to(bases[:, None, None], (NROWS, 8, 128))
    out = _sc_spline(x3, coeff_flat, bases3)
    return out.reshape(x.shape)
